# baseline (speedup 1.0000x reference)
import functools

import jax
import jax.numpy as jnp
from jax import lax
from jax.experimental import pallas as pl
from jax.experimental.pallas import tpu as pltpu
from jax.experimental.pallas import tpu_sc as plsc

N = 100000
D = 128
NWIDTH = 16
H = 128
G = 512

TILE = 1024
NB = -(-N // TILE)
NPADDED = NB * TILE
NVALID_LAST = N - (NB - 1) * TILE

CH = 128
N2 = 100096
NCHUNK = N2 // CH
NC = 2
NS = 16
NWORK = NC * NS
KMAX = -(-NCHUNK // NWORK)
GROWS = G // NS


def _pre_body(x_ref, v_ref, W1_ref, b1_ref, W2_ref, b2_ref, h2_ref):
    i = pl.program_id(0)
    xv = x_ref[...]
    v = v_ref[...]
    inv = jnp.sqrt(v[:, 0:16] ** 2 + v[:, 16:32] ** 2 + v[:, 32:48] ** 2)
    h = (jax.lax.dot(xv.astype(jnp.bfloat16), W1_ref[0:D, :].astype(jnp.bfloat16),
                     preferred_element_type=jnp.float32)
         + jax.lax.dot(inv.astype(jnp.bfloat16),
                       W1_ref[D:D + NWIDTH, :].astype(jnp.bfloat16),
                       preferred_element_type=jnp.float32)
         + b1_ref[...])
    h = h * jax.nn.sigmoid(h)
    h = jax.lax.dot(h.astype(jnp.bfloat16), W2_ref[...].astype(jnp.bfloat16),
                    preferred_element_type=jnp.float32) + b2_ref[...]

    @pl.when(i < NB - 1)
    def _store():
        h2_ref[...] = h

    @pl.when(i == NB - 1)
    def _store_masked():
        ridx = lax.broadcasted_iota(jnp.int32, (TILE, 1), 0)
        h2_ref[...] = jnp.where(ridx < NVALID_LAST, h, 0.0)


def _segsum_body(h2_hbm, batch_hbm, out_hbm, rows_v, idx_v, stage_v, acc_sh):
    cid = lax.axis_index("c")
    sid = lax.axis_index("s")
    wid = sid * NC + cid

    for r in range(GROWS):
        for j in range(H // 16):
            stage_v[r, pl.ds(j * 16, 16)] = jnp.zeros((16,), jnp.float32)
    pltpu.sync_copy(stage_v, acc_sh.at[pl.ds(sid * GROWS, GROWS)])
    plsc.subcore_barrier()

    for k in range(KMAX):
        c = wid + NWORK * k

        @pl.when(c < NCHUNK)
        def _chunk():
            base = c * CH
            pltpu.sync_copy(batch_hbm.at[pl.ds(base, CH)], idx_v.at[k])
            pltpu.sync_copy(h2_hbm.at[pl.ds(base, CH), :], rows_v)
            pltpu.sync_copy(rows_v, acc_sh.at[idx_v.at[k]], add=True)

    plsc.subcore_barrier()
    pltpu.sync_copy(acc_sh.at[pl.ds(sid * GROWS, GROWS)],
                    out_hbm.at[cid, pl.ds(sid * GROWS, GROWS)])


def _post_body(pp_ref, W3_ref, b3_ref, W4_ref, b4_ref, out_ref):
    p = pp_ref[0] + pp_ref[1]
    g = jax.lax.dot(p, W3_ref[...], preferred_element_type=jnp.float32) + b3_ref[...]
    g = g * jax.nn.sigmoid(g)
    out_ref[...] = (jax.lax.dot(g, W4_ref[...], preferred_element_type=jnp.float32)
                    + b4_ref[...])


@jax.jit
def kernel(x, V, batch, W1, b1, W2, b2, W3, b3, W4, b4):
    v2 = V.reshape(N, 3 * NWIDTH)
    batch_p = jnp.concatenate(
        [batch, jnp.zeros((NPADDED - N,), jnp.int32)]).reshape(NB, 1, TILE)

    full = lambda *s: pl.BlockSpec(s, lambda i: (0,) * len(s))
    h2 = pl.pallas_call(
        _pre_body,
        grid=(NB,),
        in_specs=[
            pl.BlockSpec((TILE, D), lambda i: (i, 0)),
            pl.BlockSpec((TILE, 3 * NWIDTH), lambda i: (i, 0)),
            full(D + NWIDTH, H),
            full(1, H),
            full(H, H),
            full(1, H),
        ],
        out_specs=pl.BlockSpec((TILE, H), lambda i: (i, 0)),
        out_shape=jax.ShapeDtypeStruct((N2, H), jnp.float32),
        compiler_params=pltpu.CompilerParams(
            dimension_semantics=("arbitrary",),
        ),
    )(x, v2, W1, b1.reshape(1, H), W2, b2.reshape(1, H))

    mesh = plsc.VectorSubcoreMesh(core_axis_name="c", subcore_axis_name="s")
    segsum = functools.partial(
        pl.kernel,
        mesh=mesh,
        out_type=jax.ShapeDtypeStruct((NC, G, H), jnp.float32),
        scratch_types=[
            pltpu.VMEM((CH, H), jnp.float32),
            pltpu.VMEM((KMAX, CH), jnp.int32),
            pltpu.VMEM((GROWS, H), jnp.float32),
            pltpu.VMEM_SHARED((G, H), jnp.float32),
        ],
    )(_segsum_body)
    pooled2 = segsum(h2, batch_p.reshape(NPADDED))

    out = pl.pallas_call(
        _post_body,
        in_specs=[
            pl.BlockSpec((NC, G, H), lambda: (0, 0, 0)),
            pl.BlockSpec((H, H), lambda: (0, 0)),
            pl.BlockSpec((1, H), lambda: (0, 0)),
            pl.BlockSpec((H, 1), lambda: (0, 0)),
            pl.BlockSpec((1, 1), lambda: (0, 0)),
        ],
        out_specs=pl.BlockSpec((G, 1), lambda: (0, 0)),
        out_shape=jax.ShapeDtypeStruct((G, 1), jnp.float32),
    )(pooled2, W3, b3.reshape(1, H), W4, b4.reshape(1, 1))
    return out
